# Initial kernel scaffold; baseline (speedup 1.0000x reference)
import jax
import jax.numpy as jnp
from jax.experimental import pallas as pl


def kernel(x, edge_index, W0a, W0b, W0c, W1a, W1b, W1c):
    raise NotImplementedError("write your pallas kernel here")


# speedup vs baseline: 39.9603x; 39.9603x over previous
import jax
import jax.numpy as jnp
from jax import lax
from jax.experimental import pallas as pl
from jax.experimental.pallas import tpu as pltpu
from jax.experimental.pallas import tpu_sc as plsc

N = 100000
E = 3200000
F = 16
NPAD = 102400
DUMMY = N

NC, NS = 2, 16
NW = NC * NS
RPS = NPAD // NS

KI = 16
CHUNK = KI * 128
PER_TILE = ((E + NW * CHUNK - 1) // (NW * CHUNK)) * CHUNK
E_PAD = PER_TILE * NW
ROWS_PER_TILE = PER_TILE // 128
N_OUTER = PER_TILE // CHUNK

_mesh = plsc.VectorSubcoreMesh(core_axis_name="c", subcore_axis_name="s")


def _gs_body(table_hbm, row_hbm, col_hbm, zeros_hbm, out_hbm,
             acc_sp, idxr_v, idxc_v, msg_v, sem):
    cid = lax.axis_index("c")
    sid = lax.axis_index("s")
    wid = cid * NS + sid
    pltpu.sync_copy(zeros_hbm.at[pl.ds(sid * RPS, RPS)],
                    acc_sp.at[pl.ds(sid * RPS, RPS)])
    plsc.subcore_barrier()
    base = wid * ROWS_PER_TILE

    def outer(g, carry):
        pltpu.sync_copy(row_hbm.at[pl.ds(base + g * KI, KI)], idxr_v)
        pltpu.sync_copy(col_hbm.at[pl.ds(base + g * KI, KI)], idxc_v)
        for j in range(KI):
            pltpu.async_copy(table_hbm.at[idxr_v.at[j]], msg_v, sem).wait()
            pltpu.sync_copy(msg_v, acc_sp.at[idxc_v.at[j]], add=True)
        return carry

    lax.fori_loop(0, N_OUTER, outer, 0)
    plsc.subcore_barrier()
    pltpu.sync_copy(acc_sp.at[pl.ds(sid * RPS, RPS)],
                    out_hbm.at[cid, pl.ds(sid * RPS, RPS)])


def _sc_gather_scatter(table, row2d, col2d, zeros_nf):
    return pl.kernel(
        _gs_body,
        out_type=jax.ShapeDtypeStruct((NC, NPAD, F), jnp.float32),
        mesh=_mesh,
        scratch_types=[
            pltpu.VMEM_SHARED((NPAD, F), jnp.float32),
            pltpu.VMEM((KI, 128), jnp.int32),
            pltpu.VMEM((KI, 128), jnp.int32),
            pltpu.VMEM((128, F), jnp.float32),
            pltpu.SemaphoreType.DMA,
        ],
        compiler_params=pltpu.CompilerParams(use_tc_tiling_on_sc=False),
    )(table, row2d, col2d, zeros_nf)


def _leaky(v):
    return jnp.where(v > 0, v, 0.01 * v)


def _mlp(v, Wa, Wb, Wc):
    h = _leaky(jnp.dot(v, Wa, preferred_element_type=jnp.float32))
    h = _leaky(jnp.dot(h, Wb, preferred_element_type=jnp.float32))
    return jnp.dot(h, Wc, preferred_element_type=jnp.float32)


def _dinv_of(degab):
    deg = degab[0, :, 0] + degab[1, :, 0]
    safe = jnp.where(deg > 0, deg, 1.0)
    return jnp.where(deg > 0, lax.rsqrt(safe), 0.0)


def _stageA_body(deg_ref, x_ref, wa, wb, wc, tab_ref):
    dinv = _dinv_of(deg_ref[...])

    h0 = _mlp(x_ref[...], wa[...], wb[...], wc[...])
    lane = lax.broadcasted_iota(jnp.int32, h0.shape, 1)
    tab_ref[...] = dinv[:, None] * h0 + jnp.where(lane == 3, dinv[:, None], 0.0)


def _stageB_body(deg_ref, x_ref, ab_ref, w0a, w0b, w0c, w1a, w1b, w1c,
                 verts_ref, tab_ref):
    dinv = _dinv_of(deg_ref[...])
    S = ab_ref[0] + ab_ref[1]
    s = S[:, 3]
    h0 = _mlp(x_ref[...], w0a[...], w0b[...], w0c[...])
    verts1 = x_ref[...] + dinv[:, None] * S - h0 * (dinv * s)[:, None]
    h1 = _mlp(verts1, w1a[...], w1b[...], w1c[...])
    verts_ref[...] = verts1
    tab_ref[...] = dinv[:, None] * h1


def _stageC_body(deg_ref, ab0_ref, verts_ref, ab1_ref, w1a, w1b, w1c, out_ref):
    dinv = _dinv_of(deg_ref[...])
    s = (ab0_ref[0] + ab0_ref[1])[:, 3]
    A1 = ab1_ref[0] + ab1_ref[1]
    verts1 = verts_ref[...]
    h1 = _mlp(verts1, w1a[...], w1b[...], w1c[...])
    out_ref[...] = verts1 + dinv[:, None] * A1 - h1 * (dinv * s)[:, None]


BLK = 1024
GRID = NPAD // BLK

_nf_spec = pl.BlockSpec((BLK, F), lambda i: (i, 0))
_ab_spec = pl.BlockSpec((NC, BLK, F), lambda i: (0, i, 0))
_deg_spec = _ab_spec


def _w_spec(shape):
    return pl.BlockSpec(shape, lambda i: tuple(0 for _ in shape))


def _tc_stageA(deg2, x_pad, W0a, W0b, W0c):
    return pl.pallas_call(
        _stageA_body,
        grid=(GRID,),
        in_specs=[_deg_spec, _nf_spec,
                  _w_spec((F, 16)), _w_spec((16, 16)), _w_spec((16, F))],
        out_specs=_nf_spec,
        out_shape=jax.ShapeDtypeStruct((NPAD, F), jnp.float32),
    )(deg2, x_pad, W0a, W0b, W0c)


def _tc_stageB(deg2, x_pad, AB0, W0a, W0b, W0c, W1a, W1b, W1c):
    return pl.pallas_call(
        _stageB_body,
        grid=(GRID,),
        in_specs=[_deg_spec, _nf_spec, _ab_spec,
                  _w_spec((F, 16)), _w_spec((16, 16)), _w_spec((16, F)),
                  _w_spec((F, 16)), _w_spec((16, 16)), _w_spec((16, F))],
        out_specs=[_nf_spec, _nf_spec],
        out_shape=[jax.ShapeDtypeStruct((NPAD, F), jnp.float32),
                   jax.ShapeDtypeStruct((NPAD, F), jnp.float32)],
    )(deg2, x_pad, AB0, W0a, W0b, W0c, W1a, W1b, W1c)


def _tc_stageC(deg2, AB0, verts1, AB1, W1a, W1b, W1c):
    return pl.pallas_call(
        _stageC_body,
        grid=(GRID,),
        in_specs=[_deg_spec, _ab_spec, _nf_spec, _ab_spec,
                  _w_spec((F, 16)), _w_spec((16, 16)), _w_spec((16, F))],
        out_specs=_nf_spec,
        out_shape=jax.ShapeDtypeStruct((NPAD, F), jnp.float32),
    )(deg2, AB0, verts1, AB1, W1a, W1b, W1c)


def kernel(x, edge_index, W0a, W0b, W0c, W1a, W1b, W1c):
    x_pad = jnp.pad(x, ((0, NPAD - N), (0, F - 3)))
    W0a_p = jnp.pad(W0a, ((0, F - 3), (0, 0)))
    W0c_p = jnp.pad(W0c, ((0, 0), (0, F - 3)))
    W1a_p = jnp.pad(W1a, ((0, F - 3), (0, 0)))
    W1c_p = jnp.pad(W1c, ((0, 0), (0, F - 3)))

    row = jnp.pad(edge_index[0], (0, E_PAD - E), constant_values=DUMMY)
    col = jnp.pad(edge_index[1], (0, E_PAD - E), constant_values=DUMMY)
    row2d = row.reshape(E_PAD // 128, 128)
    col2d = col.reshape(E_PAD // 128, 128)

    zeros_nf = jnp.zeros((NPAD, F), jnp.float32)
    e0_table = jnp.zeros((NPAD, F), jnp.float32).at[:, 0].set(1.0)

    deg2 = _sc_gather_scatter(e0_table, row2d, col2d, zeros_nf)
    table0 = _tc_stageA(deg2, x_pad, W0a_p, W0b, W0c_p)
    AB0 = _sc_gather_scatter(table0, row2d, col2d, zeros_nf)
    verts1, table1 = _tc_stageB(deg2, x_pad, AB0, W0a_p, W0b, W0c_p,
                                W1a_p, W1b, W1c_p)
    AB1 = _sc_gather_scatter(table1, row2d, col2d, zeros_nf)
    out = _tc_stageC(deg2, AB0, verts1, AB1, W1a_p, W1b, W1c_p)
    return out[:N, :3]


# speedup vs baseline: 72.6403x; 1.8178x over previous
import jax
import jax.numpy as jnp
from jax import lax
from jax.experimental import pallas as pl
from jax.experimental.pallas import tpu as pltpu
from jax.experimental.pallas import tpu_sc as plsc

N = 100000
E = 3200000
F = 16
NPAD = 102400
DUMMY = N

NC, NS = 2, 16
NW = NC * NS
RPS = NPAD // NS

KI = 8
CHUNK = KI * 128
PER_TILE = ((E + NW * CHUNK - 1) // (NW * CHUNK)) * CHUNK
E_PAD = PER_TILE * NW
ROWS_PER_TILE = PER_TILE // 128
N_OUTER = PER_TILE // CHUNK

_mesh = plsc.VectorSubcoreMesh(core_axis_name="c", subcore_axis_name="s")


def _gs_body(table_hbm, row_hbm, col_hbm, zeros_hbm, out_hbm,
             acc_sp, idxr_v, idxc_v, msg_v, gsem, ssem):
    cid = lax.axis_index("c")
    sid = lax.axis_index("s")
    wid = cid * NS + sid
    pltpu.sync_copy(zeros_hbm.at[pl.ds(sid * RPS, RPS)],
                    acc_sp.at[pl.ds(sid * RPS, RPS)])
    plsc.subcore_barrier()
    base = wid * ROWS_PER_TILE

    def outer(g, carry):
        pltpu.sync_copy(row_hbm.at[pl.ds(base + g * KI, KI)], idxr_v)
        pltpu.sync_copy(col_hbm.at[pl.ds(base + g * KI, KI)], idxc_v)
        gds = []
        for j in range(KI):
            d = pltpu.make_async_copy(table_hbm.at[idxr_v.at[j]],
                                      msg_v.at[j], gsem)
            d.start()
            gds.append(d)
        for d in gds:
            d.wait()
        sds = []
        for j in range(KI):
            d = pltpu.make_async_copy(msg_v.at[j],
                                      acc_sp.at[idxc_v.at[j]], ssem)
            d.start(add=True)
            sds.append(d)
        for d in sds:
            d.wait()
        return carry

    lax.fori_loop(0, N_OUTER, outer, 0)
    plsc.subcore_barrier()
    pltpu.sync_copy(acc_sp.at[pl.ds(sid * RPS, RPS)],
                    out_hbm.at[cid, pl.ds(sid * RPS, RPS)])


def _sc_gather_scatter(table, row2d, col2d, zeros_nf):
    return pl.kernel(
        _gs_body,
        out_type=jax.ShapeDtypeStruct((NC, NPAD, F), jnp.float32),
        mesh=_mesh,
        scratch_types=[
            pltpu.VMEM_SHARED((NPAD, F), jnp.float32),
            pltpu.VMEM((KI, 128), jnp.int32),
            pltpu.VMEM((KI, 128), jnp.int32),
            pltpu.VMEM((KI, 128, F), jnp.float32),
            pltpu.SemaphoreType.DMA,
            pltpu.SemaphoreType.DMA,
        ],
        compiler_params=pltpu.CompilerParams(use_tc_tiling_on_sc=False),
    )(table, row2d, col2d, zeros_nf)


def _deg_body(col_hbm, zeros_hbm, ones_hbm, out_hbm,
              acc_sp, idxc_v, ones_v, ssem):
    cid = lax.axis_index("c")
    sid = lax.axis_index("s")
    wid = cid * NS + sid
    pltpu.sync_copy(zeros_hbm.at[pl.ds(sid * RPS, RPS)],
                    acc_sp.at[pl.ds(sid * RPS, RPS)])
    pltpu.sync_copy(ones_hbm, ones_v)
    plsc.subcore_barrier()
    base = wid * ROWS_PER_TILE

    def outer(g, carry):
        pltpu.sync_copy(col_hbm.at[pl.ds(base + g * KI, KI)], idxc_v)
        sds = []
        for j in range(KI):
            d = pltpu.make_async_copy(ones_v, acc_sp.at[idxc_v.at[j]], ssem)
            d.start(add=True)
            sds.append(d)
        for d in sds:
            d.wait()
        return carry

    lax.fori_loop(0, N_OUTER, outer, 0)
    plsc.subcore_barrier()
    pltpu.sync_copy(acc_sp.at[pl.ds(sid * RPS, RPS)],
                    out_hbm.at[cid, pl.ds(sid * RPS, RPS)])


def _sc_deg(col2d, zeros_nf, ones_nf):
    return pl.kernel(
        _deg_body,
        out_type=jax.ShapeDtypeStruct((NC, NPAD, F), jnp.float32),
        mesh=_mesh,
        scratch_types=[
            pltpu.VMEM_SHARED((NPAD, F), jnp.float32),
            pltpu.VMEM((KI, 128), jnp.int32),
            pltpu.VMEM((128, F), jnp.float32),
            pltpu.SemaphoreType.DMA,
        ],
        compiler_params=pltpu.CompilerParams(use_tc_tiling_on_sc=False),
    )(col2d, zeros_nf, ones_nf)


def _leaky(v):
    return jnp.where(v > 0, v, 0.01 * v)


def _mlp(v, Wa, Wb, Wc):
    h = _leaky(jnp.dot(v, Wa, preferred_element_type=jnp.float32))
    h = _leaky(jnp.dot(h, Wb, preferred_element_type=jnp.float32))
    return jnp.dot(h, Wc, preferred_element_type=jnp.float32)


def _dinv_of(degab):
    deg = degab[0, :, 0] + degab[1, :, 0]
    safe = jnp.where(deg > 0, deg, 1.0)
    return jnp.where(deg > 0, lax.rsqrt(safe), 0.0)


def _stageA_body(deg_ref, x_ref, wa, wb, wc, tab_ref):
    dinv = _dinv_of(deg_ref[...])

    h0 = _mlp(x_ref[...], wa[...], wb[...], wc[...])
    lane = lax.broadcasted_iota(jnp.int32, h0.shape, 1)
    tab_ref[...] = dinv[:, None] * h0 + jnp.where(lane == 3, dinv[:, None], 0.0)


def _stageB_body(deg_ref, x_ref, ab_ref, w0a, w0b, w0c, w1a, w1b, w1c,
                 verts_ref, tab_ref):
    dinv = _dinv_of(deg_ref[...])
    S = ab_ref[0] + ab_ref[1]
    s = S[:, 3]
    h0 = _mlp(x_ref[...], w0a[...], w0b[...], w0c[...])
    verts1 = x_ref[...] + dinv[:, None] * S - h0 * (dinv * s)[:, None]
    h1 = _mlp(verts1, w1a[...], w1b[...], w1c[...])
    verts_ref[...] = verts1
    tab_ref[...] = dinv[:, None] * h1


def _stageC_body(deg_ref, ab0_ref, verts_ref, ab1_ref, w1a, w1b, w1c, out_ref):
    dinv = _dinv_of(deg_ref[...])
    s = (ab0_ref[0] + ab0_ref[1])[:, 3]
    A1 = ab1_ref[0] + ab1_ref[1]
    verts1 = verts_ref[...]
    h1 = _mlp(verts1, w1a[...], w1b[...], w1c[...])
    out_ref[...] = verts1 + dinv[:, None] * A1 - h1 * (dinv * s)[:, None]


BLK = 1024
GRID = NPAD // BLK

_nf_spec = pl.BlockSpec((BLK, F), lambda i: (i, 0))
_ab_spec = pl.BlockSpec((NC, BLK, F), lambda i: (0, i, 0))
_deg_spec = _ab_spec


def _w_spec(shape):
    return pl.BlockSpec(shape, lambda i: tuple(0 for _ in shape))


def _tc_stageA(deg2, x_pad, W0a, W0b, W0c):
    return pl.pallas_call(
        _stageA_body,
        grid=(GRID,),
        in_specs=[_deg_spec, _nf_spec,
                  _w_spec((F, 16)), _w_spec((16, 16)), _w_spec((16, F))],
        out_specs=_nf_spec,
        out_shape=jax.ShapeDtypeStruct((NPAD, F), jnp.float32),
    )(deg2, x_pad, W0a, W0b, W0c)


def _tc_stageB(deg2, x_pad, AB0, W0a, W0b, W0c, W1a, W1b, W1c):
    return pl.pallas_call(
        _stageB_body,
        grid=(GRID,),
        in_specs=[_deg_spec, _nf_spec, _ab_spec,
                  _w_spec((F, 16)), _w_spec((16, 16)), _w_spec((16, F)),
                  _w_spec((F, 16)), _w_spec((16, 16)), _w_spec((16, F))],
        out_specs=[_nf_spec, _nf_spec],
        out_shape=[jax.ShapeDtypeStruct((NPAD, F), jnp.float32),
                   jax.ShapeDtypeStruct((NPAD, F), jnp.float32)],
    )(deg2, x_pad, AB0, W0a, W0b, W0c, W1a, W1b, W1c)


def _tc_stageC(deg2, AB0, verts1, AB1, W1a, W1b, W1c):
    return pl.pallas_call(
        _stageC_body,
        grid=(GRID,),
        in_specs=[_deg_spec, _ab_spec, _nf_spec, _ab_spec,
                  _w_spec((F, 16)), _w_spec((16, 16)), _w_spec((16, F))],
        out_specs=_nf_spec,
        out_shape=jax.ShapeDtypeStruct((NPAD, F), jnp.float32),
    )(deg2, AB0, verts1, AB1, W1a, W1b, W1c)


def kernel(x, edge_index, W0a, W0b, W0c, W1a, W1b, W1c):
    x_pad = jnp.pad(x, ((0, NPAD - N), (0, F - 3)))
    W0a_p = jnp.pad(W0a, ((0, F - 3), (0, 0)))
    W0c_p = jnp.pad(W0c, ((0, 0), (0, F - 3)))
    W1a_p = jnp.pad(W1a, ((0, F - 3), (0, 0)))
    W1c_p = jnp.pad(W1c, ((0, 0), (0, F - 3)))

    row = jnp.pad(edge_index[0], (0, E_PAD - E), constant_values=DUMMY)
    col = jnp.pad(edge_index[1], (0, E_PAD - E), constant_values=DUMMY)
    row2d = row.reshape(E_PAD // 128, 128)
    col2d = col.reshape(E_PAD // 128, 128)

    zeros_nf = jnp.zeros((NPAD, F), jnp.float32)
    ones128 = jnp.zeros((128, F), jnp.float32).at[:, 0].set(1.0)

    deg2 = _sc_deg(col2d, zeros_nf, ones128)
    table0 = _tc_stageA(deg2, x_pad, W0a_p, W0b, W0c_p)
    AB0 = _sc_gather_scatter(table0, row2d, col2d, zeros_nf)
    verts1, table1 = _tc_stageB(deg2, x_pad, AB0, W0a_p, W0b, W0c_p,
                                W1a_p, W1b, W1c_p)
    AB1 = _sc_gather_scatter(table1, row2d, col2d, zeros_nf)
    out = _tc_stageC(deg2, AB0, verts1, AB1, W1a_p, W1b, W1c_p)
    return out[:N, :3]


# speedup vs baseline: 77.7624x; 1.0705x over previous
import jax
import jax.numpy as jnp
from jax import lax
from jax.experimental import pallas as pl
from jax.experimental.pallas import tpu as pltpu
from jax.experimental.pallas import tpu_sc as plsc

N = 100000
E = 3200000
F = 8
NPAD = 102400
DUMMY = N

NC, NS = 2, 16
NW = NC * NS
RPS = NPAD // NS

KI = 8
CHUNK = KI * 128
PER_TILE = ((E + NW * CHUNK - 1) // (NW * CHUNK)) * CHUNK
E_PAD = PER_TILE * NW
ROWS_PER_TILE = PER_TILE // 128
N_OUTER = PER_TILE // CHUNK

_mesh = plsc.VectorSubcoreMesh(core_axis_name="c", subcore_axis_name="s")


def _gs_body(table_hbm, row_hbm, col_hbm, zeros_hbm, out_hbm,
             acc_sp, idxr_v, idxc_v, msg_v, gsem, ssem):
    cid = lax.axis_index("c")
    sid = lax.axis_index("s")
    wid = cid * NS + sid
    pltpu.sync_copy(zeros_hbm.at[pl.ds(sid * RPS, RPS)],
                    acc_sp.at[pl.ds(sid * RPS, RPS)])
    plsc.subcore_barrier()
    base = wid * ROWS_PER_TILE

    def outer(g, carry):
        pltpu.sync_copy(row_hbm.at[pl.ds(base + g * KI, KI)], idxr_v)
        pltpu.sync_copy(col_hbm.at[pl.ds(base + g * KI, KI)], idxc_v)
        gds = []
        for j in range(KI):
            d = pltpu.make_async_copy(table_hbm.at[idxr_v.at[j]],
                                      msg_v.at[j], gsem)
            d.start()
            gds.append(d)
        for d in gds:
            d.wait()
        sds = []
        for j in range(KI):
            d = pltpu.make_async_copy(msg_v.at[j],
                                      acc_sp.at[idxc_v.at[j]], ssem)
            d.start(add=True)
            sds.append(d)
        for d in sds:
            d.wait()
        return carry

    lax.fori_loop(0, N_OUTER, outer, 0)
    plsc.subcore_barrier()
    pltpu.sync_copy(acc_sp.at[pl.ds(sid * RPS, RPS)],
                    out_hbm.at[cid, pl.ds(sid * RPS, RPS)])


def _sc_gather_scatter(table, row2d, col2d, zeros_nf):
    return pl.kernel(
        _gs_body,
        out_type=jax.ShapeDtypeStruct((NC, NPAD, F), jnp.float32),
        mesh=_mesh,
        scratch_types=[
            pltpu.VMEM_SHARED((NPAD, F), jnp.float32),
            pltpu.VMEM((KI, 128), jnp.int32),
            pltpu.VMEM((KI, 128), jnp.int32),
            pltpu.VMEM((KI, 128, F), jnp.float32),
            pltpu.SemaphoreType.DMA,
            pltpu.SemaphoreType.DMA,
        ],
        compiler_params=pltpu.CompilerParams(use_tc_tiling_on_sc=False),
    )(table, row2d, col2d, zeros_nf)


def _deg_body(col_hbm, zeros_hbm, ones_hbm, out_hbm,
              acc_sp, idxc_v, ones_v, ssem):
    cid = lax.axis_index("c")
    sid = lax.axis_index("s")
    wid = cid * NS + sid
    pltpu.sync_copy(zeros_hbm.at[pl.ds(sid * RPS, RPS)],
                    acc_sp.at[pl.ds(sid * RPS, RPS)])
    pltpu.sync_copy(ones_hbm, ones_v)
    plsc.subcore_barrier()
    base = wid * ROWS_PER_TILE

    def outer(g, carry):
        pltpu.sync_copy(col_hbm.at[pl.ds(base + g * KI, KI)], idxc_v)
        sds = []
        for j in range(KI):
            d = pltpu.make_async_copy(ones_v, acc_sp.at[idxc_v.at[j]], ssem)
            d.start(add=True)
            sds.append(d)
        for d in sds:
            d.wait()
        return carry

    lax.fori_loop(0, N_OUTER, outer, 0)
    plsc.subcore_barrier()
    pltpu.sync_copy(acc_sp.at[pl.ds(sid * RPS, RPS)],
                    out_hbm.at[cid, pl.ds(sid * RPS, RPS)])


def _sc_deg(col2d, zeros_nf, ones_nf):
    return pl.kernel(
        _deg_body,
        out_type=jax.ShapeDtypeStruct((NC, NPAD, F), jnp.float32),
        mesh=_mesh,
        scratch_types=[
            pltpu.VMEM_SHARED((NPAD, F), jnp.float32),
            pltpu.VMEM((KI, 128), jnp.int32),
            pltpu.VMEM((128, F), jnp.float32),
            pltpu.SemaphoreType.DMA,
        ],
        compiler_params=pltpu.CompilerParams(use_tc_tiling_on_sc=False),
    )(col2d, zeros_nf, ones_nf)


def _leaky(v):
    return jnp.where(v > 0, v, 0.01 * v)


def _mlp(v, Wa, Wb, Wc):
    h = _leaky(jnp.dot(v, Wa, preferred_element_type=jnp.float32))
    h = _leaky(jnp.dot(h, Wb, preferred_element_type=jnp.float32))
    return jnp.dot(h, Wc, preferred_element_type=jnp.float32)


def _dinv_of(degab):
    deg = degab[0, :, 0] + degab[1, :, 0]
    safe = jnp.where(deg > 0, deg, 1.0)
    return jnp.where(deg > 0, lax.rsqrt(safe), 0.0)


def _stageA_body(deg_ref, x_ref, wa, wb, wc, tab_ref):
    dinv = _dinv_of(deg_ref[...])

    h0 = _mlp(x_ref[...], wa[...], wb[...], wc[...])
    lane = lax.broadcasted_iota(jnp.int32, h0.shape, 1)
    tab_ref[...] = dinv[:, None] * h0 + jnp.where(lane == 3, dinv[:, None], 0.0)


def _stageB_body(deg_ref, x_ref, ab_ref, w0a, w0b, w0c, w1a, w1b, w1c,
                 verts_ref, tab_ref):
    dinv = _dinv_of(deg_ref[...])
    S = ab_ref[0] + ab_ref[1]
    s = S[:, 3]
    h0 = _mlp(x_ref[...], w0a[...], w0b[...], w0c[...])
    verts1 = x_ref[...] + dinv[:, None] * S - h0 * (dinv * s)[:, None]
    h1 = _mlp(verts1, w1a[...], w1b[...], w1c[...])
    verts_ref[...] = verts1
    tab_ref[...] = dinv[:, None] * h1


def _stageC_body(deg_ref, ab0_ref, verts_ref, ab1_ref, w1a, w1b, w1c, out_ref):
    dinv = _dinv_of(deg_ref[...])
    s = (ab0_ref[0] + ab0_ref[1])[:, 3]
    A1 = ab1_ref[0] + ab1_ref[1]
    verts1 = verts_ref[...]
    h1 = _mlp(verts1, w1a[...], w1b[...], w1c[...])
    out_ref[...] = verts1 + dinv[:, None] * A1 - h1 * (dinv * s)[:, None]


BLK = 1024
GRID = NPAD // BLK

_nf_spec = pl.BlockSpec((BLK, F), lambda i: (i, 0))
_ab_spec = pl.BlockSpec((NC, BLK, F), lambda i: (0, i, 0))
_deg_spec = _ab_spec


def _w_spec(shape):
    return pl.BlockSpec(shape, lambda i: tuple(0 for _ in shape))


def _tc_stageA(deg2, x_pad, W0a, W0b, W0c):
    return pl.pallas_call(
        _stageA_body,
        grid=(GRID,),
        in_specs=[_deg_spec, _nf_spec,
                  _w_spec((F, 16)), _w_spec((16, 16)), _w_spec((16, F))],
        out_specs=_nf_spec,
        out_shape=jax.ShapeDtypeStruct((NPAD, F), jnp.float32),
    )(deg2, x_pad, W0a, W0b, W0c)


def _tc_stageB(deg2, x_pad, AB0, W0a, W0b, W0c, W1a, W1b, W1c):
    return pl.pallas_call(
        _stageB_body,
        grid=(GRID,),
        in_specs=[_deg_spec, _nf_spec, _ab_spec,
                  _w_spec((F, 16)), _w_spec((16, 16)), _w_spec((16, F)),
                  _w_spec((F, 16)), _w_spec((16, 16)), _w_spec((16, F))],
        out_specs=[_nf_spec, _nf_spec],
        out_shape=[jax.ShapeDtypeStruct((NPAD, F), jnp.float32),
                   jax.ShapeDtypeStruct((NPAD, F), jnp.float32)],
    )(deg2, x_pad, AB0, W0a, W0b, W0c, W1a, W1b, W1c)


def _tc_stageC(deg2, AB0, verts1, AB1, W1a, W1b, W1c):
    return pl.pallas_call(
        _stageC_body,
        grid=(GRID,),
        in_specs=[_deg_spec, _ab_spec, _nf_spec, _ab_spec,
                  _w_spec((F, 16)), _w_spec((16, 16)), _w_spec((16, F))],
        out_specs=_nf_spec,
        out_shape=jax.ShapeDtypeStruct((NPAD, F), jnp.float32),
    )(deg2, AB0, verts1, AB1, W1a, W1b, W1c)


def kernel(x, edge_index, W0a, W0b, W0c, W1a, W1b, W1c):
    x_pad = jnp.pad(x, ((0, NPAD - N), (0, F - 3)))
    W0a_p = jnp.pad(W0a, ((0, F - 3), (0, 0)))
    W0c_p = jnp.pad(W0c, ((0, 0), (0, F - 3)))
    W1a_p = jnp.pad(W1a, ((0, F - 3), (0, 0)))
    W1c_p = jnp.pad(W1c, ((0, 0), (0, F - 3)))

    row = jnp.pad(edge_index[0], (0, E_PAD - E), constant_values=DUMMY)
    col = jnp.pad(edge_index[1], (0, E_PAD - E), constant_values=DUMMY)
    row2d = row.reshape(E_PAD // 128, 128)
    col2d = col.reshape(E_PAD // 128, 128)

    zeros_nf = jnp.zeros((NPAD, F), jnp.float32)
    ones128 = jnp.zeros((128, F), jnp.float32).at[:, 0].set(1.0)

    deg2 = _sc_deg(col2d, zeros_nf, ones128)
    table0 = _tc_stageA(deg2, x_pad, W0a_p, W0b, W0c_p)
    AB0 = _sc_gather_scatter(table0, row2d, col2d, zeros_nf)
    verts1, table1 = _tc_stageB(deg2, x_pad, AB0, W0a_p, W0b, W0c_p,
                                W1a_p, W1b, W1c_p)
    AB1 = _sc_gather_scatter(table1, row2d, col2d, zeros_nf)
    out = _tc_stageC(deg2, AB0, verts1, AB1, W1a_p, W1b, W1c_p)
    return out[:N, :3]


# speedup vs baseline: 90.1186x; 1.1589x over previous
import jax
import jax.numpy as jnp
from jax import lax
from jax.experimental import pallas as pl
from jax.experimental.pallas import tpu as pltpu
from jax.experimental.pallas import tpu_sc as plsc

N = 100000
E = 3200000
F = 8
NPAD = 102400
DUMMY = N

NC, NS = 2, 16
NW = NC * NS
RPS = NPAD // NS

KI = 8
CHUNK = KI * 128
PER_TILE = ((E + NW * CHUNK - 1) // (NW * CHUNK)) * CHUNK
E_PAD = PER_TILE * NW
ROWS_PER_TILE = PER_TILE // 128
N_OUTER = PER_TILE // CHUNK

_mesh = plsc.VectorSubcoreMesh(core_axis_name="c", subcore_axis_name="s")


def _gather_start(table_hbm, idx_v, msg_v, bi, bm, gsem):
    for j in range(KI):
        pltpu.make_async_copy(table_hbm.at[idx_v.at[bi, j, 0]],
                              msg_v.at[bm, j], gsem).start()


def _gather_drain(table_hbm, idx_v, msg_v, bi, bm, gsem):
    for j in range(KI):
        pltpu.make_async_copy(table_hbm.at[idx_v.at[bi, j, 0]],
                              msg_v.at[bm, j], gsem).wait()


def _scatter_start(acc_sp, idx_v, msg_v, bi, bm, ssem):
    for j in range(KI):
        pltpu.make_async_copy(msg_v.at[bm, j],
                              acc_sp.at[idx_v.at[bi, j, 1]], ssem).start(add=True)


def _scatter_drain(acc_sp, idx_v, msg_v, bi, bm, ssem):
    for j in range(KI):
        pltpu.make_async_copy(msg_v.at[bm, j],
                              acc_sp.at[idx_v.at[bi, j, 1]], ssem).wait()


def _gs_body(table_hbm, rc_hbm, zeros_hbm, out_hbm,
             acc_sp, idx_v, msg_v, gsem, ssem, isem):
    cid = lax.axis_index("c")
    sid = lax.axis_index("s")
    wid = cid * NS + sid
    pltpu.sync_copy(zeros_hbm.at[pl.ds(sid * RPS, RPS)],
                    acc_sp.at[pl.ds(sid * RPS, RPS)])
    plsc.subcore_barrier()
    base = wid * ROWS_PER_TILE
    pltpu.make_async_copy(rc_hbm.at[pl.ds(base, KI)], idx_v.at[0], isem).start()

    def step(g, carry):
        bm = lax.rem(g, 2)
        bi = lax.rem(g, 3)
        bn = lax.rem(g + 1, 3)

        @pl.when(g >= 2)
        def _():
            _scatter_drain(acc_sp, idx_v, msg_v, bi, bm, ssem)

        pltpu.make_async_copy(rc_hbm.at[pl.ds(base + g * KI, KI)],
                              idx_v.at[bi], isem).wait()
        _gather_start(table_hbm, idx_v, msg_v, bi, bm, gsem)
        pltpu.make_async_copy(rc_hbm.at[pl.ds(base + (g + 1) * KI, KI)],
                              idx_v.at[bn], isem).start()
        _gather_drain(table_hbm, idx_v, msg_v, bi, bm, gsem)
        _scatter_start(acc_sp, idx_v, msg_v, bi, bm, ssem)
        return carry

    lax.fori_loop(0, N_OUTER, step, 0)
    for bm in (N_OUTER % 2, 1 - (N_OUTER % 2)):
        _scatter_drain(acc_sp, idx_v, msg_v, 0, bm, ssem)
    pltpu.make_async_copy(rc_hbm.at[pl.ds(base, KI)],
                          idx_v.at[0], isem).wait()
    plsc.subcore_barrier()
    pltpu.sync_copy(acc_sp.at[pl.ds(sid * RPS, RPS)],
                    out_hbm.at[cid, pl.ds(sid * RPS, RPS)])


def _sc_gather_scatter(table, rc2d, zeros_nf):
    return pl.kernel(
        _gs_body,
        out_type=jax.ShapeDtypeStruct((NC, NPAD, F), jnp.float32),
        mesh=_mesh,
        scratch_types=[
            pltpu.VMEM_SHARED((NPAD, F), jnp.float32),
            pltpu.VMEM((3, KI, 2, 128), jnp.int32),
            pltpu.VMEM((2, KI, 128, F), jnp.float32),
            pltpu.SemaphoreType.DMA,
            pltpu.SemaphoreType.DMA,
            pltpu.SemaphoreType.DMA,
        ],
        compiler_params=pltpu.CompilerParams(use_tc_tiling_on_sc=False),
    )(table, rc2d, zeros_nf)


def _ones_scatter_start(acc_sp, idx_v, ones_v, bi, ssem):
    for j in range(KI):
        pltpu.make_async_copy(ones_v,
                              acc_sp.at[idx_v.at[bi, j, 1]], ssem).start(add=True)


def _ones_scatter_drain(acc_sp, idx_v, ones_v, bi, ssem):
    for j in range(KI):
        pltpu.make_async_copy(ones_v,
                              acc_sp.at[idx_v.at[bi, j, 1]], ssem).wait()


def _deg_body(rc_hbm, zeros_hbm, ones_hbm, out_hbm,
              acc_sp, idx_v, ones_v, ssem, isem):
    cid = lax.axis_index("c")
    sid = lax.axis_index("s")
    wid = cid * NS + sid
    pltpu.sync_copy(zeros_hbm.at[pl.ds(sid * RPS, RPS)],
                    acc_sp.at[pl.ds(sid * RPS, RPS)])
    pltpu.sync_copy(ones_hbm, ones_v)
    plsc.subcore_barrier()
    base = wid * ROWS_PER_TILE
    pltpu.make_async_copy(rc_hbm.at[pl.ds(base, KI)], idx_v.at[0], isem).start()

    def step(g, carry):
        bi = lax.rem(g, 3)
        bn = lax.rem(g + 1, 3)

        @pl.when(g >= 2)
        def _():
            _ones_scatter_drain(acc_sp, idx_v, ones_v, bi, ssem)

        pltpu.make_async_copy(rc_hbm.at[pl.ds(base + g * KI, KI)],
                              idx_v.at[bi], isem).wait()
        pltpu.make_async_copy(rc_hbm.at[pl.ds(base + (g + 1) * KI, KI)],
                              idx_v.at[bn], isem).start()
        _ones_scatter_start(acc_sp, idx_v, ones_v, bi, ssem)
        return carry

    lax.fori_loop(0, N_OUTER, step, 0)
    for _ in range(2):
        _ones_scatter_drain(acc_sp, idx_v, ones_v, 0, ssem)
    pltpu.make_async_copy(rc_hbm.at[pl.ds(base, KI)],
                          idx_v.at[0], isem).wait()
    plsc.subcore_barrier()
    pltpu.sync_copy(acc_sp.at[pl.ds(sid * RPS, RPS)],
                    out_hbm.at[cid, pl.ds(sid * RPS, RPS)])


def _sc_deg(rc2d, zeros_nf, ones_nf):
    return pl.kernel(
        _deg_body,
        out_type=jax.ShapeDtypeStruct((NC, NPAD, F), jnp.float32),
        mesh=_mesh,
        scratch_types=[
            pltpu.VMEM_SHARED((NPAD, F), jnp.float32),
            pltpu.VMEM((3, KI, 2, 128), jnp.int32),
            pltpu.VMEM((128, F), jnp.float32),
            pltpu.SemaphoreType.DMA,
            pltpu.SemaphoreType.DMA,
        ],
        compiler_params=pltpu.CompilerParams(use_tc_tiling_on_sc=False),
    )(rc2d, zeros_nf, ones_nf)


def _leaky(v):
    return jnp.where(v > 0, v, 0.01 * v)


def _mlp(v, Wa, Wb, Wc):
    h = _leaky(jnp.dot(v, Wa, preferred_element_type=jnp.float32))
    h = _leaky(jnp.dot(h, Wb, preferred_element_type=jnp.float32))
    return jnp.dot(h, Wc, preferred_element_type=jnp.float32)


def _dinv_of(degab):
    deg = degab[0, :, 0] + degab[1, :, 0]
    safe = jnp.where(deg > 0, deg, 1.0)
    return jnp.where(deg > 0, lax.rsqrt(safe), 0.0)


def _stageA_body(deg_ref, x_ref, wa, wb, wc, tab_ref):
    dinv = _dinv_of(deg_ref[...])

    h0 = _mlp(x_ref[...], wa[...], wb[...], wc[...])
    lane = lax.broadcasted_iota(jnp.int32, h0.shape, 1)
    tab_ref[...] = dinv[:, None] * h0 + jnp.where(lane == 3, dinv[:, None], 0.0)


def _stageB_body(deg_ref, x_ref, ab_ref, w0a, w0b, w0c, w1a, w1b, w1c,
                 verts_ref, tab_ref):
    dinv = _dinv_of(deg_ref[...])
    S = ab_ref[0] + ab_ref[1]
    s = S[:, 3]
    h0 = _mlp(x_ref[...], w0a[...], w0b[...], w0c[...])
    verts1 = x_ref[...] + dinv[:, None] * S - h0 * (dinv * s)[:, None]
    h1 = _mlp(verts1, w1a[...], w1b[...], w1c[...])
    verts_ref[...] = verts1
    tab_ref[...] = dinv[:, None] * h1


def _stageC_body(deg_ref, ab0_ref, verts_ref, ab1_ref, w1a, w1b, w1c, out_ref):
    dinv = _dinv_of(deg_ref[...])
    s = (ab0_ref[0] + ab0_ref[1])[:, 3]
    A1 = ab1_ref[0] + ab1_ref[1]
    verts1 = verts_ref[...]
    h1 = _mlp(verts1, w1a[...], w1b[...], w1c[...])
    out_ref[...] = verts1 + dinv[:, None] * A1 - h1 * (dinv * s)[:, None]


BLK = 1024
GRID = NPAD // BLK

_nf_spec = pl.BlockSpec((BLK, F), lambda i: (i, 0))
_ab_spec = pl.BlockSpec((NC, BLK, F), lambda i: (0, i, 0))
_deg_spec = _ab_spec


def _w_spec(shape):
    return pl.BlockSpec(shape, lambda i: tuple(0 for _ in shape))


def _tc_stageA(deg2, x_pad, W0a, W0b, W0c):
    return pl.pallas_call(
        _stageA_body,
        grid=(GRID,),
        in_specs=[_deg_spec, _nf_spec,
                  _w_spec((F, 16)), _w_spec((16, 16)), _w_spec((16, F))],
        out_specs=_nf_spec,
        out_shape=jax.ShapeDtypeStruct((NPAD, F), jnp.float32),
    )(deg2, x_pad, W0a, W0b, W0c)


def _tc_stageB(deg2, x_pad, AB0, W0a, W0b, W0c, W1a, W1b, W1c):
    return pl.pallas_call(
        _stageB_body,
        grid=(GRID,),
        in_specs=[_deg_spec, _nf_spec, _ab_spec,
                  _w_spec((F, 16)), _w_spec((16, 16)), _w_spec((16, F)),
                  _w_spec((F, 16)), _w_spec((16, 16)), _w_spec((16, F))],
        out_specs=[_nf_spec, _nf_spec],
        out_shape=[jax.ShapeDtypeStruct((NPAD, F), jnp.float32),
                   jax.ShapeDtypeStruct((NPAD, F), jnp.float32)],
    )(deg2, x_pad, AB0, W0a, W0b, W0c, W1a, W1b, W1c)


def _tc_stageC(deg2, AB0, verts1, AB1, W1a, W1b, W1c):
    return pl.pallas_call(
        _stageC_body,
        grid=(GRID,),
        in_specs=[_deg_spec, _ab_spec, _nf_spec, _ab_spec,
                  _w_spec((F, 16)), _w_spec((16, 16)), _w_spec((16, F))],
        out_specs=_nf_spec,
        out_shape=jax.ShapeDtypeStruct((NPAD, F), jnp.float32),
    )(deg2, AB0, verts1, AB1, W1a, W1b, W1c)


def kernel(x, edge_index, W0a, W0b, W0c, W1a, W1b, W1c):
    x_pad = jnp.pad(x, ((0, NPAD - N), (0, F - 3)))
    W0a_p = jnp.pad(W0a, ((0, F - 3), (0, 0)))
    W0c_p = jnp.pad(W0c, ((0, 0), (0, F - 3)))
    W1a_p = jnp.pad(W1a, ((0, F - 3), (0, 0)))
    W1c_p = jnp.pad(W1c, ((0, 0), (0, F - 3)))

    row = jnp.pad(edge_index[0], (0, E_PAD - E), constant_values=DUMMY)
    col = jnp.pad(edge_index[1], (0, E_PAD - E), constant_values=DUMMY)
    rc = jnp.stack([row.reshape(E_PAD // 128, 128),
                    col.reshape(E_PAD // 128, 128)], axis=1)
    rc2d = jnp.pad(rc, ((0, KI), (0, 0), (0, 0)))

    zeros_nf = jnp.zeros((NPAD, F), jnp.float32)
    ones128 = jnp.zeros((128, F), jnp.float32).at[:, 0].set(1.0)

    deg2 = _sc_deg(rc2d, zeros_nf, ones128)
    table0 = _tc_stageA(deg2, x_pad, W0a_p, W0b, W0c_p)
    AB0 = _sc_gather_scatter(table0, rc2d, zeros_nf)
    verts1, table1 = _tc_stageB(deg2, x_pad, AB0, W0a_p, W0b, W0c_p,
                                W1a_p, W1b, W1c_p)
    AB1 = _sc_gather_scatter(table1, rc2d, zeros_nf)
    out = _tc_stageC(deg2, AB0, verts1, AB1, W1a_p, W1b, W1c_p)
    return out[:N, :3]


# speedup vs baseline: 116.6590x; 1.2945x over previous
import jax
import jax.numpy as jnp
from jax import lax
from jax.experimental import pallas as pl
from jax.experimental.pallas import tpu as pltpu
from jax.experimental.pallas import tpu_sc as plsc

N = 100000
E = 3200000
F = 8
NPAD = 102400
DUMMY = N

NC, NS = 2, 16
NW = NC * NS
RPS = NPAD // NS

KI = 8
CHUNK = KI * 128
PER_TILE = ((E + NW * CHUNK - 1) // (NW * CHUNK)) * CHUNK
E_PAD = PER_TILE * NW
ROWS_PER_TILE = PER_TILE // 128
N_OUTER = PER_TILE // CHUNK

_mesh = plsc.VectorSubcoreMesh(core_axis_name="c", subcore_axis_name="s")


def _gather_start(table_hbm, idx_v, msg_v, bi, bm, gsem):
    for j in range(KI):
        pltpu.make_async_copy(table_hbm.at[idx_v.at[bi, j, 0]],
                              msg_v.at[bm, j], gsem).start()


def _gather_drain(table_hbm, idx_v, msg_v, bi, bm, gsem):
    for j in range(KI):
        pltpu.make_async_copy(table_hbm.at[idx_v.at[bi, j, 0]],
                              msg_v.at[bm, j], gsem).wait()


def _scatter_start(acc_sp, idx_v, msg_v, bi, bm, ssem):
    for j in range(KI):
        pltpu.make_async_copy(msg_v.at[bm, j],
                              acc_sp.at[idx_v.at[bi, j, 1]], ssem).start(add=True)


def _scatter_drain(acc_sp, idx_v, msg_v, bi, bm, ssem):
    for j in range(KI):
        pltpu.make_async_copy(msg_v.at[bm, j],
                              acc_sp.at[idx_v.at[bi, j, 1]], ssem).wait()


def _gs_body(table_hbm, rc_hbm, zeros_hbm, out_hbm,
             acc_sp, idx_v, msg_v, gsem, ssem, isem):
    cid = lax.axis_index("c")
    sid = lax.axis_index("s")
    wid = cid * NS + sid
    pltpu.sync_copy(zeros_hbm.at[pl.ds(sid * RPS, RPS)],
                    acc_sp.at[pl.ds(sid * RPS, RPS)])
    plsc.subcore_barrier()
    base = wid * ROWS_PER_TILE
    pltpu.make_async_copy(rc_hbm.at[pl.ds(base, KI)], idx_v.at[0], isem).start()

    def step(g, carry):
        bm = lax.rem(g, 2)
        bi = lax.rem(g, 3)
        bn = lax.rem(g + 1, 3)

        @pl.when(g >= 2)
        def _():
            _scatter_drain(acc_sp, idx_v, msg_v, bi, bm, ssem)

        pltpu.make_async_copy(rc_hbm.at[pl.ds(base + g * KI, KI)],
                              idx_v.at[bi], isem).wait()
        _gather_start(table_hbm, idx_v, msg_v, bi, bm, gsem)
        pltpu.make_async_copy(rc_hbm.at[pl.ds(base + (g + 1) * KI, KI)],
                              idx_v.at[bn], isem).start()
        _gather_drain(table_hbm, idx_v, msg_v, bi, bm, gsem)
        _scatter_start(acc_sp, idx_v, msg_v, bi, bm, ssem)
        return carry

    lax.fori_loop(0, N_OUTER, step, 0)
    for bm in (N_OUTER % 2, 1 - (N_OUTER % 2)):
        _scatter_drain(acc_sp, idx_v, msg_v, 0, bm, ssem)
    pltpu.make_async_copy(rc_hbm.at[pl.ds(base, KI)],
                          idx_v.at[0], isem).wait()
    plsc.subcore_barrier()
    pltpu.sync_copy(acc_sp.at[pl.ds(sid * RPS, RPS)],
                    out_hbm.at[cid, pl.ds(sid * RPS, RPS)])


def _sc_gather_scatter(table, rc2d, zeros_nf):
    return pl.kernel(
        _gs_body,
        out_type=jax.ShapeDtypeStruct((NC, NPAD, F), jnp.float32),
        mesh=_mesh,
        scratch_types=[
            pltpu.VMEM_SHARED((NPAD, F), jnp.float32),
            pltpu.VMEM((3, KI, 2, 128), jnp.int32),
            pltpu.VMEM((2, KI, 128, F), jnp.float32),
            pltpu.SemaphoreType.DMA,
            pltpu.SemaphoreType.DMA,
            pltpu.SemaphoreType.DMA,
        ],
        compiler_params=pltpu.CompilerParams(use_tc_tiling_on_sc=False),
    )(table, rc2d, zeros_nf)


def _ones_scatter_start(acc_sp, idx_v, ones_v, bi, ssem):
    for j in range(KI):
        pltpu.make_async_copy(ones_v,
                              acc_sp.at[idx_v.at[bi, j, 1]], ssem).start(add=True)


def _ones_scatter_drain(acc_sp, idx_v, ones_v, bi, ssem):
    for j in range(KI):
        pltpu.make_async_copy(ones_v,
                              acc_sp.at[idx_v.at[bi, j, 1]], ssem).wait()


def _deg_body(rc_hbm, zeros_hbm, ones_hbm, out_hbm,
              acc_sp, idx_v, ones_v, ssem, isem):
    cid = lax.axis_index("c")
    sid = lax.axis_index("s")
    wid = cid * NS + sid
    pltpu.sync_copy(zeros_hbm.at[pl.ds(sid * RPS, RPS)],
                    acc_sp.at[pl.ds(sid * RPS, RPS)])
    pltpu.sync_copy(ones_hbm, ones_v)
    plsc.subcore_barrier()
    base = wid * ROWS_PER_TILE
    pltpu.make_async_copy(rc_hbm.at[pl.ds(base, KI)], idx_v.at[0], isem).start()

    def step(g, carry):
        bi = lax.rem(g, 3)
        bn = lax.rem(g + 1, 3)

        @pl.when(g >= 2)
        def _():
            _ones_scatter_drain(acc_sp, idx_v, ones_v, bi, ssem)

        pltpu.make_async_copy(rc_hbm.at[pl.ds(base + g * KI, KI)],
                              idx_v.at[bi], isem).wait()
        pltpu.make_async_copy(rc_hbm.at[pl.ds(base + (g + 1) * KI, KI)],
                              idx_v.at[bn], isem).start()
        _ones_scatter_start(acc_sp, idx_v, ones_v, bi, ssem)
        return carry

    lax.fori_loop(0, N_OUTER, step, 0)
    for _ in range(2):
        _ones_scatter_drain(acc_sp, idx_v, ones_v, 0, ssem)
    pltpu.make_async_copy(rc_hbm.at[pl.ds(base, KI)],
                          idx_v.at[0], isem).wait()
    plsc.subcore_barrier()
    pltpu.sync_copy(acc_sp.at[pl.ds(sid * RPS, RPS)],
                    out_hbm.at[cid, pl.ds(sid * RPS, RPS)])


def _sc_deg(rc2d, zeros_nf, ones_nf):
    return pl.kernel(
        _deg_body,
        out_type=jax.ShapeDtypeStruct((NC, NPAD, F), jnp.float32),
        mesh=_mesh,
        scratch_types=[
            pltpu.VMEM_SHARED((NPAD, F), jnp.float32),
            pltpu.VMEM((3, KI, 2, 128), jnp.int32),
            pltpu.VMEM((128, F), jnp.float32),
            pltpu.SemaphoreType.DMA,
            pltpu.SemaphoreType.DMA,
        ],
        compiler_params=pltpu.CompilerParams(use_tc_tiling_on_sc=False),
    )(rc2d, zeros_nf, ones_nf)


def _leaky(v):
    return jnp.where(v > 0, v, 0.01 * v)


def _mlp_t(v, WaT, WbT, WcT):
    h = _leaky(jnp.dot(WaT, v, preferred_element_type=jnp.float32))
    h = _leaky(jnp.dot(WbT, h, preferred_element_type=jnp.float32))
    return jnp.dot(WcT, h, preferred_element_type=jnp.float32)


def _dinv_row(degc):
    deg = degc[0] + degc[1]
    safe = jnp.where(deg > 0, deg, 1.0)
    return jnp.where(deg > 0, lax.rsqrt(safe), 0.0)[None, :]


def _stageA_body(degc_ref, xT_ref, wa, wb, wc, tabT_ref):
    dinv = _dinv_row(degc_ref[...])
    h0 = _mlp_t(xT_ref[...], wa[...], wb[...], wc[...])
    r = lax.broadcasted_iota(jnp.int32, h0.shape, 0)
    tabT_ref[...] = dinv * h0 + jnp.where(r == 3, dinv, 0.0)


def _stageB_body(degc_ref, xT_ref, abT_ref, w0a, w0b, w0c, w1a, w1b, w1c,
                 vertsT_ref, sd_ref, tabT_ref):
    dinv = _dinv_row(degc_ref[...])
    ST = abT_ref[0] + abT_ref[1]
    sd = dinv * ST[3:4]
    h0 = _mlp_t(xT_ref[...], w0a[...], w0b[...], w0c[...])
    verts1 = xT_ref[...] + dinv * ST - h0 * sd
    h1 = _mlp_t(verts1, w1a[...], w1b[...], w1c[...])
    vertsT_ref[...] = verts1
    sd_ref[...] = sd
    tabT_ref[...] = dinv * h1


def _stageC_body(degc_ref, vertsT_ref, sd_ref, abT_ref, w1a, w1b, w1c,
                 outT_ref):
    dinv = _dinv_row(degc_ref[...])
    A1 = abT_ref[0] + abT_ref[1]
    verts1 = vertsT_ref[...]
    h1 = _mlp_t(verts1, w1a[...], w1b[...], w1c[...])
    outT_ref[...] = verts1 + dinv * A1 - h1 * sd_ref[...]


BLKN = 12800
GRID = NPAD // BLKN

_degc_spec = pl.BlockSpec((NC, BLKN), lambda i: (0, i))
_fT_spec = pl.BlockSpec((F, BLKN), lambda i: (0, i))
_abT_spec = pl.BlockSpec((NC, F, BLKN), lambda i: (0, 0, i))
_sd_spec = pl.BlockSpec((1, BLKN), lambda i: (0, i))


def _w_spec(shape):
    return pl.BlockSpec(shape, lambda i: tuple(0 for _ in shape))


def _tc_stageA(degc, xT, W0aT, W0bT, W0cT):
    return pl.pallas_call(
        _stageA_body,
        grid=(GRID,),
        in_specs=[_degc_spec, _fT_spec,
                  _w_spec((16, F)), _w_spec((16, 16)), _w_spec((F, 16))],
        out_specs=_fT_spec,
        out_shape=jax.ShapeDtypeStruct((F, NPAD), jnp.float32),
    )(degc, xT, W0aT, W0bT, W0cT)


def _tc_stageB(degc, xT, AB0T, W0aT, W0bT, W0cT, W1aT, W1bT, W1cT):
    return pl.pallas_call(
        _stageB_body,
        grid=(GRID,),
        in_specs=[_degc_spec, _fT_spec, _abT_spec,
                  _w_spec((16, F)), _w_spec((16, 16)), _w_spec((F, 16)),
                  _w_spec((16, F)), _w_spec((16, 16)), _w_spec((F, 16))],
        out_specs=[_fT_spec, _sd_spec, _fT_spec],
        out_shape=[jax.ShapeDtypeStruct((F, NPAD), jnp.float32),
                   jax.ShapeDtypeStruct((1, NPAD), jnp.float32),
                   jax.ShapeDtypeStruct((F, NPAD), jnp.float32)],
    )(degc, xT, AB0T, W0aT, W0bT, W0cT, W1aT, W1bT, W1cT)


def _tc_stageC(degc, vertsT, sd, AB1T, W1aT, W1bT, W1cT):
    return pl.pallas_call(
        _stageC_body,
        grid=(GRID,),
        in_specs=[_degc_spec, _fT_spec, _sd_spec, _abT_spec,
                  _w_spec((16, F)), _w_spec((16, 16)), _w_spec((F, 16))],
        out_specs=_fT_spec,
        out_shape=jax.ShapeDtypeStruct((F, NPAD), jnp.float32),
    )(degc, vertsT, sd, AB1T, W1aT, W1bT, W1cT)


def kernel(x, edge_index, W0a, W0b, W0c, W1a, W1b, W1c):
    xT = jnp.pad(x.T, ((0, F - 3), (0, NPAD - N)))
    W0aT = jnp.pad(W0a, ((0, F - 3), (0, 0))).T
    W0bT = W0b.T
    W0cT = jnp.pad(W0c, ((0, 0), (0, F - 3))).T
    W1aT = jnp.pad(W1a, ((0, F - 3), (0, 0))).T
    W1bT = W1b.T
    W1cT = jnp.pad(W1c, ((0, 0), (0, F - 3))).T

    row = jnp.pad(edge_index[0], (0, E_PAD - E), constant_values=DUMMY)
    col = jnp.pad(edge_index[1], (0, E_PAD - E), constant_values=DUMMY)
    rc = jnp.stack([row.reshape(E_PAD // 128, 128),
                    col.reshape(E_PAD // 128, 128)], axis=1)
    rc2d = jnp.pad(rc, ((0, KI), (0, 0), (0, 0)))

    zeros_nf = jnp.zeros((NPAD, F), jnp.float32)
    ones128 = jnp.zeros((128, F), jnp.float32).at[:, 0].set(1.0)

    deg2 = _sc_deg(rc2d, zeros_nf, ones128)
    degc = deg2[:, :, 0]
    table0T = _tc_stageA(degc, xT, W0aT, W0bT, W0cT)
    AB0 = _sc_gather_scatter(table0T.T, rc2d, zeros_nf)
    vertsT, sd, table1T = _tc_stageB(degc, xT, AB0.transpose(0, 2, 1),
                                     W0aT, W0bT, W0cT,
                                     W1aT, W1bT, W1cT)
    AB1 = _sc_gather_scatter(table1T.T, rc2d, zeros_nf)
    outT = _tc_stageC(degc, vertsT, sd, AB1.transpose(0, 2, 1),
                      W1aT, W1bT, W1cT)
    return outT[:3, :N].T


# speedup vs baseline: 121.5593x; 1.0420x over previous
import jax
import jax.numpy as jnp
from jax import lax
from jax.experimental import pallas as pl
from jax.experimental.pallas import tpu as pltpu
from jax.experimental.pallas import tpu_sc as plsc

N = 100000
E = 3200000
F = 8
NPAD = 102400
DUMMY = N

NC, NS = 2, 16
NW = NC * NS
RPS = NPAD // NS

KI = 8
CHUNK = KI * 128
PER_TILE = ((E + NW * CHUNK - 1) // (NW * CHUNK)) * CHUNK
E_PAD = PER_TILE * NW
ROWS_PER_TILE = PER_TILE // 128
N_OUTER = PER_TILE // CHUNK

_mesh = plsc.VectorSubcoreMesh(core_axis_name="c", subcore_axis_name="s")


def _gather_start(table_hbm, ridx_v, msg_v, bi, bm, gsem):
    for j in range(KI):
        pltpu.make_async_copy(table_hbm.at[ridx_v.at[bi, j]],
                              msg_v.at[bm, j], gsem).start()


def _gather_drain(table_hbm, ridx_v, msg_v, bi, bm, gsem):
    for j in range(KI):
        pltpu.make_async_copy(table_hbm.at[ridx_v.at[bi, j]],
                              msg_v.at[bm, j], gsem).wait()


def _scatter_start(acc_sp, cidx_v, msg_v, bi, bm, ssem):
    for j in range(KI):
        pltpu.make_async_copy(msg_v.at[bm, j],
                              acc_sp.at[cidx_v.at[bi, j]], ssem).start(add=True)


def _scatter_drain(acc_sp, cidx_v, msg_v, bi, bm, ssem):
    for j in range(KI):
        pltpu.make_async_copy(msg_v.at[bm, j],
                              acc_sp.at[cidx_v.at[bi, j]], ssem).wait()


def _gs_body(table_hbm, row_hbm, col_hbm, zeros_hbm, out_hbm,
             acc_sp, ridx_v, cidx_v, msg_v, gsem, ssem, irsem, icsem):
    cid = lax.axis_index("c")
    sid = lax.axis_index("s")
    wid = cid * NS + sid
    pltpu.sync_copy(zeros_hbm.at[pl.ds(sid * RPS, RPS)],
                    acc_sp.at[pl.ds(sid * RPS, RPS)])
    plsc.subcore_barrier()
    base = wid * ROWS_PER_TILE
    pltpu.make_async_copy(row_hbm.at[pl.ds(base, KI)], ridx_v.at[0], irsem).start()
    pltpu.make_async_copy(col_hbm.at[pl.ds(base, KI)], cidx_v.at[0], icsem).start()

    def step(g, carry):
        bm = lax.rem(g, 2)
        bi = lax.rem(g, 3)
        bn = lax.rem(g + 1, 3)

        @pl.when(g >= 2)
        def _():
            _scatter_drain(acc_sp, cidx_v, msg_v, bi, bm, ssem)

        pltpu.make_async_copy(row_hbm.at[pl.ds(base + g * KI, KI)],
                              ridx_v.at[bi], irsem).wait()
        _gather_start(table_hbm, ridx_v, msg_v, bi, bm, gsem)
        pltpu.make_async_copy(row_hbm.at[pl.ds(base + (g + 1) * KI, KI)],
                              ridx_v.at[bn], irsem).start()
        pltpu.make_async_copy(col_hbm.at[pl.ds(base + g * KI, KI)],
                              cidx_v.at[bi], icsem).wait()
        pltpu.make_async_copy(col_hbm.at[pl.ds(base + (g + 1) * KI, KI)],
                              cidx_v.at[bn], icsem).start()
        _gather_drain(table_hbm, ridx_v, msg_v, bi, bm, gsem)
        _scatter_start(acc_sp, cidx_v, msg_v, bi, bm, ssem)
        return carry

    lax.fori_loop(0, N_OUTER, step, 0)
    for bm in (N_OUTER % 2, 1 - (N_OUTER % 2)):
        _scatter_drain(acc_sp, cidx_v, msg_v, 0, bm, ssem)
    pltpu.make_async_copy(row_hbm.at[pl.ds(base, KI)],
                          ridx_v.at[0], irsem).wait()
    pltpu.make_async_copy(col_hbm.at[pl.ds(base, KI)],
                          cidx_v.at[0], icsem).wait()
    plsc.subcore_barrier()
    pltpu.sync_copy(acc_sp.at[pl.ds(sid * RPS, RPS)],
                    out_hbm.at[cid, pl.ds(sid * RPS, RPS)])


def _sc_gather_scatter(table, row2d, col2d, zeros_nf):
    return pl.kernel(
        _gs_body,
        out_type=jax.ShapeDtypeStruct((NC, NPAD, F), jnp.float32),
        mesh=_mesh,
        scratch_types=[
            pltpu.VMEM_SHARED((NPAD, F), jnp.float32),
            pltpu.VMEM((3, KI, 128), jnp.int32),
            pltpu.VMEM((3, KI, 128), jnp.int32),
            pltpu.VMEM((2, KI, 128, F), jnp.float32),
            pltpu.SemaphoreType.DMA,
            pltpu.SemaphoreType.DMA,
            pltpu.SemaphoreType.DMA,
            pltpu.SemaphoreType.DMA,
        ],
        compiler_params=pltpu.CompilerParams(use_tc_tiling_on_sc=False),
    )(table, row2d, col2d, zeros_nf)


def _ones_scatter_start(acc_sp, cidx_v, ones_v, bi, ssem):
    for j in range(KI):
        pltpu.make_async_copy(ones_v,
                              acc_sp.at[cidx_v.at[bi, j]], ssem).start(add=True)


def _ones_scatter_drain(acc_sp, cidx_v, ones_v, bi, ssem):
    for j in range(KI):
        pltpu.make_async_copy(ones_v,
                              acc_sp.at[cidx_v.at[bi, j]], ssem).wait()


def _deg_body(col_hbm, zeros_hbm, ones_hbm, out_hbm,
              acc_sp, cidx_v, ones_v, ssem, isem):
    cid = lax.axis_index("c")
    sid = lax.axis_index("s")
    wid = cid * NS + sid
    pltpu.sync_copy(zeros_hbm.at[pl.ds(sid * RPS, RPS)],
                    acc_sp.at[pl.ds(sid * RPS, RPS)])
    pltpu.sync_copy(ones_hbm, ones_v)
    plsc.subcore_barrier()
    base = wid * ROWS_PER_TILE
    pltpu.make_async_copy(col_hbm.at[pl.ds(base, KI)], cidx_v.at[0], isem).start()

    def step(g, carry):
        bi = lax.rem(g, 3)
        bn = lax.rem(g + 1, 3)

        @pl.when(g >= 2)
        def _():
            _ones_scatter_drain(acc_sp, cidx_v, ones_v, bi, ssem)

        pltpu.make_async_copy(col_hbm.at[pl.ds(base + g * KI, KI)],
                              cidx_v.at[bi], isem).wait()
        pltpu.make_async_copy(col_hbm.at[pl.ds(base + (g + 1) * KI, KI)],
                              cidx_v.at[bn], isem).start()
        _ones_scatter_start(acc_sp, cidx_v, ones_v, bi, ssem)
        return carry

    lax.fori_loop(0, N_OUTER, step, 0)
    for _ in range(2):
        _ones_scatter_drain(acc_sp, cidx_v, ones_v, 0, ssem)
    pltpu.make_async_copy(col_hbm.at[pl.ds(base, KI)],
                          cidx_v.at[0], isem).wait()
    plsc.subcore_barrier()
    pltpu.sync_copy(acc_sp.at[pl.ds(sid * RPS, RPS)],
                    out_hbm.at[cid, pl.ds(sid * RPS, RPS)])


def _sc_deg(col2d, zeros_nf, ones_nf):
    return pl.kernel(
        _deg_body,
        out_type=jax.ShapeDtypeStruct((NC, NPAD, F), jnp.float32),
        mesh=_mesh,
        scratch_types=[
            pltpu.VMEM_SHARED((NPAD, F), jnp.float32),
            pltpu.VMEM((3, KI, 128), jnp.int32),
            pltpu.VMEM((128, F), jnp.float32),
            pltpu.SemaphoreType.DMA,
            pltpu.SemaphoreType.DMA,
        ],
        compiler_params=pltpu.CompilerParams(use_tc_tiling_on_sc=False),
    )(col2d, zeros_nf, ones_nf)


def _leaky(v):
    return jnp.where(v > 0, v, 0.01 * v)


def _mlp_t(v, WaT, WbT, WcT):
    h = _leaky(jnp.dot(WaT, v, preferred_element_type=jnp.float32))
    h = _leaky(jnp.dot(WbT, h, preferred_element_type=jnp.float32))
    return jnp.dot(WcT, h, preferred_element_type=jnp.float32)


def _dinv_row(degc):
    deg = degc[0] + degc[1]
    safe = jnp.where(deg > 0, deg, 1.0)
    return jnp.where(deg > 0, lax.rsqrt(safe), 0.0)[None, :]


def _stageA_body(degc_ref, xT_ref, wa, wb, wc, tabT_ref):
    dinv = _dinv_row(degc_ref[...])
    h0 = _mlp_t(xT_ref[...], wa[...], wb[...], wc[...])
    r = lax.broadcasted_iota(jnp.int32, h0.shape, 0)
    tabT_ref[...] = dinv * h0 + jnp.where(r == 3, dinv, 0.0)


def _stageB_body(degc_ref, xT_ref, abT_ref, w0a, w0b, w0c, w1a, w1b, w1c,
                 vertsT_ref, sd_ref, tabT_ref):
    dinv = _dinv_row(degc_ref[...])
    ST = abT_ref[0] + abT_ref[1]
    sd = dinv * ST[3:4]
    h0 = _mlp_t(xT_ref[...], w0a[...], w0b[...], w0c[...])
    verts1 = xT_ref[...] + dinv * ST - h0 * sd
    h1 = _mlp_t(verts1, w1a[...], w1b[...], w1c[...])
    vertsT_ref[...] = verts1
    sd_ref[...] = sd
    tabT_ref[...] = dinv * h1


def _stageC_body(degc_ref, vertsT_ref, sd_ref, abT_ref, w1a, w1b, w1c,
                 outT_ref):
    dinv = _dinv_row(degc_ref[...])
    A1 = abT_ref[0] + abT_ref[1]
    verts1 = vertsT_ref[...]
    h1 = _mlp_t(verts1, w1a[...], w1b[...], w1c[...])
    outT_ref[...] = verts1 + dinv * A1 - h1 * sd_ref[...]


BLKN = 12800
GRID = NPAD // BLKN

_degc_spec = pl.BlockSpec((NC, BLKN), lambda i: (0, i))
_fT_spec = pl.BlockSpec((F, BLKN), lambda i: (0, i))
_abT_spec = pl.BlockSpec((NC, F, BLKN), lambda i: (0, 0, i))
_sd_spec = pl.BlockSpec((1, BLKN), lambda i: (0, i))


def _w_spec(shape):
    return pl.BlockSpec(shape, lambda i: tuple(0 for _ in shape))


def _tc_stageA(degc, xT, W0aT, W0bT, W0cT):
    return pl.pallas_call(
        _stageA_body,
        grid=(GRID,),
        in_specs=[_degc_spec, _fT_spec,
                  _w_spec((16, F)), _w_spec((16, 16)), _w_spec((F, 16))],
        out_specs=_fT_spec,
        out_shape=jax.ShapeDtypeStruct((F, NPAD), jnp.float32),
    )(degc, xT, W0aT, W0bT, W0cT)


def _tc_stageB(degc, xT, AB0T, W0aT, W0bT, W0cT, W1aT, W1bT, W1cT):
    return pl.pallas_call(
        _stageB_body,
        grid=(GRID,),
        in_specs=[_degc_spec, _fT_spec, _abT_spec,
                  _w_spec((16, F)), _w_spec((16, 16)), _w_spec((F, 16)),
                  _w_spec((16, F)), _w_spec((16, 16)), _w_spec((F, 16))],
        out_specs=[_fT_spec, _sd_spec, _fT_spec],
        out_shape=[jax.ShapeDtypeStruct((F, NPAD), jnp.float32),
                   jax.ShapeDtypeStruct((1, NPAD), jnp.float32),
                   jax.ShapeDtypeStruct((F, NPAD), jnp.float32)],
    )(degc, xT, AB0T, W0aT, W0bT, W0cT, W1aT, W1bT, W1cT)


def _tc_stageC(degc, vertsT, sd, AB1T, W1aT, W1bT, W1cT):
    return pl.pallas_call(
        _stageC_body,
        grid=(GRID,),
        in_specs=[_degc_spec, _fT_spec, _sd_spec, _abT_spec,
                  _w_spec((16, F)), _w_spec((16, 16)), _w_spec((F, 16))],
        out_specs=_fT_spec,
        out_shape=jax.ShapeDtypeStruct((F, NPAD), jnp.float32),
    )(degc, vertsT, sd, AB1T, W1aT, W1bT, W1cT)


def kernel(x, edge_index, W0a, W0b, W0c, W1a, W1b, W1c):
    xT = jnp.pad(x.T, ((0, F - 3), (0, NPAD - N)))
    W0aT = jnp.pad(W0a, ((0, F - 3), (0, 0))).T
    W0bT = W0b.T
    W0cT = jnp.pad(W0c, ((0, 0), (0, F - 3))).T
    W1aT = jnp.pad(W1a, ((0, F - 3), (0, 0))).T
    W1bT = W1b.T
    W1cT = jnp.pad(W1c, ((0, 0), (0, F - 3))).T

    row2d = jnp.pad(edge_index[0], (0, E_PAD + KI * 128 - E),
                    constant_values=DUMMY).reshape(-1, 128)
    col2d = jnp.pad(edge_index[1], (0, E_PAD + KI * 128 - E),
                    constant_values=DUMMY).reshape(-1, 128)

    zeros_nf = jnp.zeros((NPAD, F), jnp.float32)
    ones128 = jnp.zeros((128, F), jnp.float32).at[:, 0].set(1.0)

    deg2 = _sc_deg(col2d, zeros_nf, ones128)
    degc = deg2[:, :, 0]
    table0T = _tc_stageA(degc, xT, W0aT, W0bT, W0cT)
    AB0 = _sc_gather_scatter(table0T.T, row2d, col2d, zeros_nf)
    vertsT, sd, table1T = _tc_stageB(degc, xT, AB0.transpose(0, 2, 1),
                                     W0aT, W0bT, W0cT,
                                     W1aT, W1bT, W1cT)
    AB1 = _sc_gather_scatter(table1T.T, row2d, col2d, zeros_nf)
    outT = _tc_stageC(degc, vertsT, sd, AB1.transpose(0, 2, 1),
                      W1aT, W1bT, W1cT)
    return outT[:3, :N].T


# speedup vs baseline: 130.3651x; 1.0724x over previous
import jax
import jax.numpy as jnp
from jax import lax
from jax.experimental import pallas as pl
from jax.experimental.pallas import tpu as pltpu
from jax.experimental.pallas import tpu_sc as plsc

N = 100000
E = 3200000
F = 8
NPAD = 102400
DUMMY = N

NC, NS = 2, 16
NW = NC * NS
RPS = NPAD // NS

KI = 16
CHUNK = KI * 128
PER_TILE = ((E + NW * CHUNK - 1) // (NW * CHUNK)) * CHUNK
E_PAD = PER_TILE * NW
ROWS_PER_TILE = PER_TILE // 128
N_OUTER = PER_TILE // CHUNK

_mesh = plsc.VectorSubcoreMesh(core_axis_name="c", subcore_axis_name="s")


def _gather_start(table_hbm, ridx_v, msg_v, bi, bm, gsem):
    for j in range(KI):
        pltpu.make_async_copy(table_hbm.at[ridx_v.at[bi, j]],
                              msg_v.at[bm, j], gsem).start()


def _gather_drain(table_hbm, ridx_v, msg_v, bi, bm, gsem):
    for j in range(KI):
        pltpu.make_async_copy(table_hbm.at[ridx_v.at[bi, j]],
                              msg_v.at[bm, j], gsem).wait()


def _scatter_start(acc_sp, cidx_v, msg_v, bi, bm, ssem):
    for j in range(KI):
        pltpu.make_async_copy(msg_v.at[bm, j],
                              acc_sp.at[cidx_v.at[bi, j]], ssem).start(add=True)


def _scatter_drain(acc_sp, cidx_v, msg_v, bi, bm, ssem):
    for j in range(KI):
        pltpu.make_async_copy(msg_v.at[bm, j],
                              acc_sp.at[cidx_v.at[bi, j]], ssem).wait()


def _gs_body(table_hbm, row_hbm, col_hbm, zeros_hbm, out_hbm,
             acc_sp, ridx_v, cidx_v, msg_v, gsem, ssem, irsem, icsem):
    cid = lax.axis_index("c")
    sid = lax.axis_index("s")
    wid = cid * NS + sid
    pltpu.sync_copy(zeros_hbm.at[pl.ds(sid * RPS, RPS)],
                    acc_sp.at[pl.ds(sid * RPS, RPS)])
    plsc.subcore_barrier()
    base = wid * ROWS_PER_TILE
    pltpu.make_async_copy(row_hbm.at[pl.ds(base, KI)], ridx_v.at[0], irsem).start()
    pltpu.make_async_copy(col_hbm.at[pl.ds(base, KI)], cidx_v.at[0], icsem).start()

    def step(g, carry):
        bm = lax.rem(g, 2)
        bi = lax.rem(g, 3)
        bn = lax.rem(g + 1, 3)

        @pl.when(g >= 2)
        def _():
            _scatter_drain(acc_sp, cidx_v, msg_v, bi, bm, ssem)

        pltpu.make_async_copy(row_hbm.at[pl.ds(base + g * KI, KI)],
                              ridx_v.at[bi], irsem).wait()
        _gather_start(table_hbm, ridx_v, msg_v, bi, bm, gsem)
        pltpu.make_async_copy(row_hbm.at[pl.ds(base + (g + 1) * KI, KI)],
                              ridx_v.at[bn], irsem).start()
        pltpu.make_async_copy(col_hbm.at[pl.ds(base + g * KI, KI)],
                              cidx_v.at[bi], icsem).wait()
        pltpu.make_async_copy(col_hbm.at[pl.ds(base + (g + 1) * KI, KI)],
                              cidx_v.at[bn], icsem).start()
        _gather_drain(table_hbm, ridx_v, msg_v, bi, bm, gsem)
        _scatter_start(acc_sp, cidx_v, msg_v, bi, bm, ssem)
        return carry

    lax.fori_loop(0, N_OUTER, step, 0)
    for bm in (N_OUTER % 2, 1 - (N_OUTER % 2)):
        _scatter_drain(acc_sp, cidx_v, msg_v, 0, bm, ssem)
    pltpu.make_async_copy(row_hbm.at[pl.ds(base, KI)],
                          ridx_v.at[0], irsem).wait()
    pltpu.make_async_copy(col_hbm.at[pl.ds(base, KI)],
                          cidx_v.at[0], icsem).wait()
    plsc.subcore_barrier()
    pltpu.sync_copy(acc_sp.at[pl.ds(sid * RPS, RPS)],
                    out_hbm.at[cid, pl.ds(sid * RPS, RPS)])


def _sc_gather_scatter(table, row2d, col2d, zeros_nf):
    return pl.kernel(
        _gs_body,
        out_type=jax.ShapeDtypeStruct((NC, NPAD, F), jnp.float32),
        mesh=_mesh,
        scratch_types=[
            pltpu.VMEM_SHARED((NPAD, F), jnp.float32),
            pltpu.VMEM((3, KI, 128), jnp.int32),
            pltpu.VMEM((3, KI, 128), jnp.int32),
            pltpu.VMEM((2, KI, 128, F), jnp.float32),
            pltpu.SemaphoreType.DMA,
            pltpu.SemaphoreType.DMA,
            pltpu.SemaphoreType.DMA,
            pltpu.SemaphoreType.DMA,
        ],
        compiler_params=pltpu.CompilerParams(use_tc_tiling_on_sc=False),
    )(table, row2d, col2d, zeros_nf)


def _ones_scatter_start(acc_sp, cidx_v, ones_v, bi, ssem):
    for j in range(KI):
        pltpu.make_async_copy(ones_v,
                              acc_sp.at[cidx_v.at[bi, j]], ssem).start(add=True)


def _ones_scatter_drain(acc_sp, cidx_v, ones_v, bi, ssem):
    for j in range(KI):
        pltpu.make_async_copy(ones_v,
                              acc_sp.at[cidx_v.at[bi, j]], ssem).wait()


def _deg_body(col_hbm, zeros_hbm, ones_hbm, out_hbm,
              acc_sp, cidx_v, ones_v, ssem, isem):
    cid = lax.axis_index("c")
    sid = lax.axis_index("s")
    wid = cid * NS + sid
    pltpu.sync_copy(zeros_hbm.at[pl.ds(sid * RPS, RPS)],
                    acc_sp.at[pl.ds(sid * RPS, RPS)])
    pltpu.sync_copy(ones_hbm, ones_v)
    plsc.subcore_barrier()
    base = wid * ROWS_PER_TILE
    pltpu.make_async_copy(col_hbm.at[pl.ds(base, KI)], cidx_v.at[0], isem).start()

    def step(g, carry):
        bi = lax.rem(g, 3)
        bn = lax.rem(g + 1, 3)

        @pl.when(g >= 2)
        def _():
            _ones_scatter_drain(acc_sp, cidx_v, ones_v, bi, ssem)

        pltpu.make_async_copy(col_hbm.at[pl.ds(base + g * KI, KI)],
                              cidx_v.at[bi], isem).wait()
        pltpu.make_async_copy(col_hbm.at[pl.ds(base + (g + 1) * KI, KI)],
                              cidx_v.at[bn], isem).start()
        _ones_scatter_start(acc_sp, cidx_v, ones_v, bi, ssem)
        return carry

    lax.fori_loop(0, N_OUTER, step, 0)
    for _ in range(2):
        _ones_scatter_drain(acc_sp, cidx_v, ones_v, 0, ssem)
    pltpu.make_async_copy(col_hbm.at[pl.ds(base, KI)],
                          cidx_v.at[0], isem).wait()
    plsc.subcore_barrier()
    pltpu.sync_copy(acc_sp.at[pl.ds(sid * RPS, RPS)],
                    out_hbm.at[cid, pl.ds(sid * RPS, RPS)])


def _sc_deg(col2d, zeros_nf, ones_nf):
    return pl.kernel(
        _deg_body,
        out_type=jax.ShapeDtypeStruct((NC, NPAD, F), jnp.float32),
        mesh=_mesh,
        scratch_types=[
            pltpu.VMEM_SHARED((NPAD, F), jnp.float32),
            pltpu.VMEM((3, KI, 128), jnp.int32),
            pltpu.VMEM((128, F), jnp.float32),
            pltpu.SemaphoreType.DMA,
            pltpu.SemaphoreType.DMA,
        ],
        compiler_params=pltpu.CompilerParams(use_tc_tiling_on_sc=False),
    )(col2d, zeros_nf, ones_nf)


def _leaky(v):
    return jnp.where(v > 0, v, 0.01 * v)


def _mlp_t(v, WaT, WbT, WcT):
    h = _leaky(jnp.dot(WaT, v, preferred_element_type=jnp.float32))
    h = _leaky(jnp.dot(WbT, h, preferred_element_type=jnp.float32))
    return jnp.dot(WcT, h, preferred_element_type=jnp.float32)


def _dinv_row(degc):
    deg = degc[0] + degc[1]
    safe = jnp.where(deg > 0, deg, 1.0)
    return jnp.where(deg > 0, lax.rsqrt(safe), 0.0)[None, :]


def _stageA_body(degc_ref, xT_ref, wa, wb, wc, tabT_ref):
    dinv = _dinv_row(degc_ref[...])
    h0 = _mlp_t(xT_ref[...], wa[...], wb[...], wc[...])
    r = lax.broadcasted_iota(jnp.int32, h0.shape, 0)
    tabT_ref[...] = dinv * h0 + jnp.where(r == 3, dinv, 0.0)


def _stageB_body(degc_ref, xT_ref, abT_ref, w0a, w0b, w0c, w1a, w1b, w1c,
                 vertsT_ref, sd_ref, tabT_ref):
    dinv = _dinv_row(degc_ref[...])
    ST = abT_ref[0] + abT_ref[1]
    sd = dinv * ST[3:4]
    h0 = _mlp_t(xT_ref[...], w0a[...], w0b[...], w0c[...])
    verts1 = xT_ref[...] + dinv * ST - h0 * sd
    h1 = _mlp_t(verts1, w1a[...], w1b[...], w1c[...])
    vertsT_ref[...] = verts1
    sd_ref[...] = sd
    tabT_ref[...] = dinv * h1


def _stageC_body(degc_ref, vertsT_ref, sd_ref, abT_ref, w1a, w1b, w1c,
                 outT_ref):
    dinv = _dinv_row(degc_ref[...])
    A1 = abT_ref[0] + abT_ref[1]
    verts1 = vertsT_ref[...]
    h1 = _mlp_t(verts1, w1a[...], w1b[...], w1c[...])
    outT_ref[...] = verts1 + dinv * A1 - h1 * sd_ref[...]


BLKN = 12800
GRID = NPAD // BLKN

_degc_spec = pl.BlockSpec((NC, BLKN), lambda i: (0, i))
_fT_spec = pl.BlockSpec((F, BLKN), lambda i: (0, i))
_abT_spec = pl.BlockSpec((NC, F, BLKN), lambda i: (0, 0, i))
_sd_spec = pl.BlockSpec((1, BLKN), lambda i: (0, i))


def _w_spec(shape):
    return pl.BlockSpec(shape, lambda i: tuple(0 for _ in shape))


def _tc_stageA(degc, xT, W0aT, W0bT, W0cT):
    return pl.pallas_call(
        _stageA_body,
        grid=(GRID,),
        in_specs=[_degc_spec, _fT_spec,
                  _w_spec((16, F)), _w_spec((16, 16)), _w_spec((F, 16))],
        out_specs=_fT_spec,
        out_shape=jax.ShapeDtypeStruct((F, NPAD), jnp.float32),
    )(degc, xT, W0aT, W0bT, W0cT)


def _tc_stageB(degc, xT, AB0T, W0aT, W0bT, W0cT, W1aT, W1bT, W1cT):
    return pl.pallas_call(
        _stageB_body,
        grid=(GRID,),
        in_specs=[_degc_spec, _fT_spec, _abT_spec,
                  _w_spec((16, F)), _w_spec((16, 16)), _w_spec((F, 16)),
                  _w_spec((16, F)), _w_spec((16, 16)), _w_spec((F, 16))],
        out_specs=[_fT_spec, _sd_spec, _fT_spec],
        out_shape=[jax.ShapeDtypeStruct((F, NPAD), jnp.float32),
                   jax.ShapeDtypeStruct((1, NPAD), jnp.float32),
                   jax.ShapeDtypeStruct((F, NPAD), jnp.float32)],
    )(degc, xT, AB0T, W0aT, W0bT, W0cT, W1aT, W1bT, W1cT)


def _tc_stageC(degc, vertsT, sd, AB1T, W1aT, W1bT, W1cT):
    return pl.pallas_call(
        _stageC_body,
        grid=(GRID,),
        in_specs=[_degc_spec, _fT_spec, _sd_spec, _abT_spec,
                  _w_spec((16, F)), _w_spec((16, 16)), _w_spec((F, 16))],
        out_specs=_fT_spec,
        out_shape=jax.ShapeDtypeStruct((F, NPAD), jnp.float32),
    )(degc, vertsT, sd, AB1T, W1aT, W1bT, W1cT)


def kernel(x, edge_index, W0a, W0b, W0c, W1a, W1b, W1c):
    xT = jnp.pad(x.T, ((0, F - 3), (0, NPAD - N)))
    W0aT = jnp.pad(W0a, ((0, F - 3), (0, 0))).T
    W0bT = W0b.T
    W0cT = jnp.pad(W0c, ((0, 0), (0, F - 3))).T
    W1aT = jnp.pad(W1a, ((0, F - 3), (0, 0))).T
    W1bT = W1b.T
    W1cT = jnp.pad(W1c, ((0, 0), (0, F - 3))).T

    row2d = jnp.pad(edge_index[0], (0, E_PAD + KI * 128 - E),
                    constant_values=DUMMY).reshape(-1, 128)
    col2d = jnp.pad(edge_index[1], (0, E_PAD + KI * 128 - E),
                    constant_values=DUMMY).reshape(-1, 128)

    zeros_nf = jnp.zeros((NPAD, F), jnp.float32)
    ones128 = jnp.zeros((128, F), jnp.float32).at[:, 0].set(1.0)

    deg2 = _sc_deg(col2d, zeros_nf, ones128)
    degc = deg2[:, :, 0]
    table0T = _tc_stageA(degc, xT, W0aT, W0bT, W0cT)
    AB0 = _sc_gather_scatter(table0T.T, row2d, col2d, zeros_nf)
    vertsT, sd, table1T = _tc_stageB(degc, xT, AB0.transpose(0, 2, 1),
                                     W0aT, W0bT, W0cT,
                                     W1aT, W1bT, W1cT)
    AB1 = _sc_gather_scatter(table1T.T, row2d, col2d, zeros_nf)
    outT = _tc_stageC(degc, vertsT, sd, AB1.transpose(0, 2, 1),
                      W1aT, W1bT, W1cT)
    return outT[:3, :N].T
